# baseline (speedup 1.0000x reference)
import functools

import jax
import jax.numpy as jnp
from jax import lax
from jax.experimental import pallas as pl
from jax.experimental.pallas import tpu as pltpu
from jax.experimental.pallas import tpu_sc as plsc

_NC = 2
_NS = 16
_NW = _NC * _NS
_CHUNK = 128
_B = 128


def _sc_alpha_gather(targets, alpha_flat, n):
    b_per_w = n // _NW
    n_chunks = b_per_w // _CHUNK
    mesh = plsc.VectorSubcoreMesh(core_axis_name="c", subcore_axis_name="s")

    @functools.partial(
        pl.kernel,
        out_type=jax.ShapeDtypeStruct((n // _CHUNK, _CHUNK), jnp.float32),
        mesh=mesh,
        compiler_params=pltpu.CompilerParams(
            use_tc_tiling_on_sc=True, needs_layout_passes=False),
        scratch_types=[
            pltpu.VMEM((b_per_w,), jnp.int32),
            pltpu.VMEM((n_chunks, _CHUNK), jnp.float32),
            pltpu.SemaphoreType.DMA,
        ],
    )
    def k(tgt_hbm, alpha_hbm, avals_out, tgt_v, a_v, sem_a):
        wid = lax.axis_index("s") * _NC + lax.axis_index("c")
        base = wid * b_per_w
        pltpu.sync_copy(tgt_hbm.at[pl.ds(base, b_per_w)], tgt_v)
        a_dmas = [
            pltpu.async_copy(
                alpha_hbm.at[tgt_v.at[pl.ds(j * _CHUNK, _CHUNK)]],
                a_v.at[j], sem_a)
            for j in range(n_chunks)
        ]
        for d in a_dmas:
            d.wait()
        pltpu.sync_copy(a_v, avals_out.at[pl.ds(wid * n_chunks, n_chunks), :])

    return k(targets, alpha_flat)


def _logp_body(x_ref, t_ref, o_ref, *, c, nb):
    b = pl.program_id(0)
    tsel = lax.broadcasted_iota(jnp.int32, (_B, nb), 1) == b
    tcol = jnp.sum(jnp.where(tsel, t_ref[...], 0), axis=1,
                   keepdims=True)
    col = lax.broadcasted_iota(jnp.int32, (_B, c), 1)
    sel = jnp.sum(jnp.where(col == tcol, x_ref[...], 0.0), axis=1,
                  keepdims=True)
    lp = jnp.log(jax.nn.sigmoid(sel))

    @pl.when(b == 0)
    def _():
        o_ref[...] = jnp.zeros((_B, nb), jnp.float32)

    ocol = lax.broadcasted_iota(jnp.int32, (_B, nb), 1)
    o_ref[...] = jnp.where(ocol == b, lp, o_ref[...])


def _combine_body(a_ref, l_ref, o_ref, *, inv_n):
    p = jnp.dot(a_ref[...], l_ref[...], preferred_element_type=jnp.float32)
    r = lax.broadcasted_iota(jnp.int32, (_B, _B), 0)
    q = lax.broadcasted_iota(jnp.int32, (_B, _B), 1)
    tr = jnp.sum(jnp.where(r == q, p, 0.0))
    o_ref[...] = (-tr * inv_n).reshape(1, 1)


def kernel(inputs, targets, alpha):
    n, c = inputs.shape
    alpha_flat = alpha.reshape(-1)
    tgt = targets.astype(jnp.int32)
    nb = n // _B

    avals = _sc_alpha_gather(tgt, alpha_flat, n)

    t2t = tgt.reshape(nb, _B).T
    lps = pl.pallas_call(
        functools.partial(_logp_body, c=c, nb=nb),
        grid=(nb,),
        in_specs=[
            pl.BlockSpec((_B, c), lambda b: (b, 0)),
            pl.BlockSpec((_B, nb), lambda b: (0, 0)),
        ],
        out_specs=pl.BlockSpec((_B, nb), lambda b: (0, 0)),
        out_shape=jax.ShapeDtypeStruct((_B, nb), jnp.float32),
    )(inputs, t2t)

    out = pl.pallas_call(
        functools.partial(_combine_body, inv_n=1.0 / n),
        out_shape=jax.ShapeDtypeStruct((1, 1), jnp.float32),
    )(avals, lps)
    return out[0, 0]


# speedup vs baseline: 2.5774x; 2.5774x over previous
import functools

import jax
import jax.numpy as jnp
from jax import lax
from jax.experimental import pallas as pl
from jax.experimental.pallas import tpu as pltpu
from jax.experimental.pallas import tpu_sc as plsc

_NC = 2
_NS = 16
_NW = _NC * _NS
_L = 16
_CHUNK = 128
_RING = 4
_G = 8


def _sc_gather(xt, targets, alpha_flat, n, c):
    b_per_w = n // _NW
    n_chunks = b_per_w // _CHUNK
    n_steps = b_per_w // _L
    n_grp = c // _G
    out_sd = jax.ShapeDtypeStruct((n // _CHUNK, _CHUNK), jnp.float32)
    mesh = plsc.VectorSubcoreMesh(core_axis_name="c", subcore_axis_name="s")

    @functools.partial(
        pl.kernel,
        out_type=(out_sd, out_sd),
        mesh=mesh,
        compiler_params=pltpu.CompilerParams(
            use_tc_tiling_on_sc=True, needs_layout_passes=False),
        scratch_types=[
            pltpu.VMEM((b_per_w,), jnp.int32),
            pltpu.VMEM((n_chunks, _CHUNK), jnp.float32),
            pltpu.VMEM((n_chunks, _CHUNK), jnp.float32),
            pltpu.VMEM((_RING, _G, b_per_w), jnp.float32),
            pltpu.SemaphoreType.DMA,
            pltpu.SemaphoreType.DMA,
            pltpu.SemaphoreType.DMA,
            pltpu.SemaphoreType.DMA,
            pltpu.SemaphoreType.DMA,
        ],
    )
    def k(xt_hbm, tgt_hbm, alpha_hbm, vals_out, avals_out,
          tgt_v, a_v, vals_v, slab_v, s0, s1, s2, s3, sem_a):
        sems = (s0, s1, s2, s3)
        wid = lax.axis_index("s") * _NC + lax.axis_index("c")
        base = wid * b_per_w
        pltpu.sync_copy(tgt_hbm.at[pl.ds(base, b_per_w)], tgt_v)
        a_dmas = [
            pltpu.async_copy(
                alpha_hbm.at[tgt_v.at[pl.ds(j * _CHUNK, _CHUNK)]],
                a_v.at[j], sem_a)
            for j in range(n_chunks)
        ]
        zero16 = jnp.zeros((_L,), jnp.float32)
        for s in range(n_steps):
            vals_v[s // 8, pl.ds((s % 8) * _L, _L)] = zero16
        for b in range(_RING):
            pltpu.async_copy(
                xt_hbm.at[pl.ds(b * _G, _G), pl.ds(base, b_per_w)],
                slab_v.at[b], sems[b])
        row_iota = lax.iota(jnp.int32, _L)

        @pl.loop(0, n_grp, step=_RING)
        def _(g0):
            for b in range(_RING):
                g = g0 + b

                @pl.when(g < n_grp)
                def _():
                    pltpu.make_async_copy(
                        xt_hbm.at[pl.ds(0, _G), pl.ds(base, b_per_w)],
                        slab_v.at[b], sems[b]).wait()
                    for s in range(n_steps):
                        t16 = tgt_v[pl.ds(s * _L, _L)]
                        col16 = s * _L + row_iota
                        v16 = plsc.load_gather(
                            slab_v.at[b],
                            [jax.lax.bitwise_and(t16, _G - 1), col16])
                        hit = (t16 >> 3) == g
                        acc = vals_v[s // 8, pl.ds((s % 8) * _L, _L)]
                        vals_v[s // 8, pl.ds((s % 8) * _L, _L)] = jnp.where(
                            hit, v16, acc)
                    nxt = g + _RING

                    @pl.when(nxt < n_grp)
                    def _():
                        pltpu.async_copy(
                            xt_hbm.at[pl.ds(nxt * _G, _G),
                                      pl.ds(base, b_per_w)],
                            slab_v.at[b], sems[b])

        for d in a_dmas:
            d.wait()
        pltpu.sync_copy(vals_v, vals_out.at[pl.ds(wid * n_chunks, n_chunks), :])
        pltpu.sync_copy(a_v, avals_out.at[pl.ds(wid * n_chunks, n_chunks), :])

    return k(xt, targets, alpha_flat)


def _loss_body(v_ref, a_ref, o_ref, *, inv_n):
    p = jax.nn.sigmoid(v_ref[...])
    o_ref[...] = (-jnp.sum(a_ref[...] * jnp.log(p)) * inv_n).reshape(1, 1)


def kernel(inputs, targets, alpha):
    n, c = inputs.shape
    alpha_flat = alpha.reshape(-1)
    tgt = targets.astype(jnp.int32)
    xt = inputs.T

    vals, avals = _sc_gather(xt, tgt, alpha_flat, n, c)

    out = pl.pallas_call(
        functools.partial(_loss_body, inv_n=1.0 / n),
        out_shape=jax.ShapeDtypeStruct((1, 1), jnp.float32),
    )(vals, avals)
    return out[0, 0]


# speedup vs baseline: 2.7994x; 1.0861x over previous
import functools

import jax
import jax.numpy as jnp
from jax import lax
from jax.experimental import pallas as pl
from jax.experimental.pallas import tpu as pltpu
from jax.experimental.pallas import tpu_sc as plsc

_NC = 2
_NS = 16
_NW = _NC * _NS
_L = 16
_CHUNK = 128
_RING = 4
_H = 32


def _sc_gather(xt, targets, alpha_flat, n, c):
    b_per_w = n // _NW
    n_chunks = b_per_w // _CHUNK
    n_steps = b_per_w // _L
    n_full = c // _H
    tail_h = c - n_full * _H
    n_slab = n_full + 1
    out_sd = jax.ShapeDtypeStruct((n // _CHUNK, _CHUNK), jnp.float32)
    mesh = plsc.VectorSubcoreMesh(core_axis_name="c", subcore_axis_name="s")

    @functools.partial(
        pl.kernel,
        out_type=(out_sd, out_sd),
        mesh=mesh,
        compiler_params=pltpu.CompilerParams(
            use_tc_tiling_on_sc=True, needs_layout_passes=False),
        scratch_types=[
            pltpu.VMEM((b_per_w,), jnp.int32),
            pltpu.VMEM((n_chunks, _CHUNK), jnp.float32),
            pltpu.VMEM((n_chunks, _CHUNK), jnp.float32),
            pltpu.VMEM((_RING, _H, b_per_w), jnp.float32),
            pltpu.SemaphoreType.DMA,
            pltpu.SemaphoreType.DMA,
            pltpu.SemaphoreType.DMA,
            pltpu.SemaphoreType.DMA,
            pltpu.SemaphoreType.DMA,
        ],
    )
    def k(xt_hbm, tgt_hbm, alpha_hbm, vals_out, avals_out,
          tgt_v, a_v, vals_v, slab_v, s0, s1, s2, s3, sem_a):
        sems = (s0, s1, s2, s3)
        wid = lax.axis_index("s") * _NC + lax.axis_index("c")
        base = wid * b_per_w

        def fire(g, b):
            pltpu.async_copy(
                xt_hbm.at[pl.ds(g * _H, _H), pl.ds(base, b_per_w)],
                slab_v.at[b], sems[b])

        def fire_tail(b):
            pltpu.async_copy(
                xt_hbm.at[pl.ds(n_full * _H, tail_h), pl.ds(base, b_per_w)],
                slab_v.at[b, pl.ds(0, tail_h), :], sems[b])

        for b in range(_RING):
            fire(b, b)
        pltpu.sync_copy(tgt_hbm.at[pl.ds(base, b_per_w)], tgt_v)
        a_dmas = [
            pltpu.async_copy(
                alpha_hbm.at[tgt_v.at[pl.ds(j * _CHUNK, _CHUNK)]],
                a_v.at[j], sem_a)
            for j in range(n_chunks)
        ]
        row_iota = lax.iota(jnp.int32, _L)

        def wait_slab(b, h):
            pltpu.make_async_copy(
                xt_hbm.at[pl.ds(0, h), pl.ds(base, b_per_w)],
                slab_v.at[b] if h == _H else slab_v.at[b, pl.ds(0, h), :],
                sems[b]).wait()

        def process(g, b, tail):
            for s in range(n_steps):
                t16 = tgt_v[pl.ds(s * _L, _L)]
                col16 = s * _L + row_iota
                if tail:
                    row16 = jax.lax.bitwise_and(t16, tail_h - 1)
                    hit = (t16 >> 3) == (c - tail_h) // 8
                else:
                    row16 = jax.lax.bitwise_and(t16, _H - 1)
                    hit = (t16 >> 5) == g
                v16 = plsc.load_gather(slab_v.at[b], [row16, col16])
                dst_r = jnp.full((_L,), s // 8, jnp.int32)
                plsc.store_scatter(vals_v, [dst_r, (s % 8) * _L + row_iota],
                                   v16, mask=hit)

        n_loop = (n_full // _RING) * _RING

        @pl.loop(0, n_loop, step=_RING)
        def _(g0):
            for b in range(_RING):
                g = g0 + b
                wait_slab(b, _H)
                process(g, b, tail=False)
                nxt = g + _RING

                @pl.when(nxt < n_full)
                def _():
                    fire(nxt, b)

                @pl.when(nxt == n_full)
                def _():
                    fire_tail(b)

        for g in range(n_loop, n_full):
            b = g % _RING
            wait_slab(b, _H)
            process(g, b, tail=False)
        bt = n_full % _RING
        wait_slab(bt, tail_h)
        process(n_full, bt, tail=True)

        for d in a_dmas:
            d.wait()
        pltpu.sync_copy(vals_v, vals_out.at[pl.ds(wid * n_chunks, n_chunks), :])
        pltpu.sync_copy(a_v, avals_out.at[pl.ds(wid * n_chunks, n_chunks), :])

    return k(xt, targets, alpha_flat)


def _loss_body(v_ref, a_ref, o_ref, *, inv_n):
    p = jax.nn.sigmoid(v_ref[...])
    o_ref[...] = (-jnp.sum(a_ref[...] * jnp.log(p)) * inv_n).reshape(1, 1)


def kernel(inputs, targets, alpha):
    n, c = inputs.shape
    alpha_flat = alpha.reshape(-1)
    tgt = targets.astype(jnp.int32)
    xt = inputs.T

    vals, avals = _sc_gather(xt, tgt, alpha_flat, n, c)

    out = pl.pallas_call(
        functools.partial(_loss_body, inv_n=1.0 / n),
        out_shape=jax.ShapeDtypeStruct((1, 1), jnp.float32),
    )(vals, avals)
    return out[0, 0]


# speedup vs baseline: 2.8526x; 1.0190x over previous
import functools

import jax
import jax.numpy as jnp
from jax import lax
from jax.experimental import pallas as pl
from jax.experimental.pallas import tpu as pltpu
from jax.experimental.pallas import tpu_sc as plsc

_NC = 2
_NS = 16
_NW = _NC * _NS
_L = 16
_CHUNK = 128
_RING = 6
_H = 32


def _sc_gather(xt, targets, alpha_flat, n, c):
    b_per_w = n // _NW
    n_chunks = b_per_w // _CHUNK
    n_steps = b_per_w // _L
    n_full = c // _H
    tail_h = c - n_full * _H
    n_slab = n_full + 1
    out_sd = jax.ShapeDtypeStruct((n // _CHUNK, _CHUNK), jnp.float32)
    mesh = plsc.VectorSubcoreMesh(core_axis_name="c", subcore_axis_name="s")

    @functools.partial(
        pl.kernel,
        out_type=(out_sd, out_sd),
        mesh=mesh,
        compiler_params=pltpu.CompilerParams(
            use_tc_tiling_on_sc=True, needs_layout_passes=False),
        scratch_types=[
            pltpu.VMEM((b_per_w,), jnp.int32),
            pltpu.VMEM((n_chunks, _CHUNK), jnp.float32),
            pltpu.VMEM((n_chunks, _CHUNK), jnp.float32),
            pltpu.VMEM((_RING, _H, b_per_w), jnp.float32),
            pltpu.SemaphoreType.DMA,
            pltpu.SemaphoreType.DMA,
            pltpu.SemaphoreType.DMA,
            pltpu.SemaphoreType.DMA,
            pltpu.SemaphoreType.DMA,
            pltpu.SemaphoreType.DMA,
            pltpu.SemaphoreType.DMA,
        ],
    )
    def k(xt_hbm, tgt_hbm, alpha_hbm, vals_out, avals_out,
          tgt_v, a_v, vals_v, slab_v, s0, s1, s2, s3, s4, s5, sem_a):
        sems = (s0, s1, s2, s3, s4, s5)
        wid = lax.axis_index("s") * _NC + lax.axis_index("c")
        base = wid * b_per_w

        def fire(g, b):
            pltpu.async_copy(
                xt_hbm.at[pl.ds(g * _H, _H), pl.ds(base, b_per_w)],
                slab_v.at[b], sems[b])

        def fire_tail(b):
            pltpu.async_copy(
                xt_hbm.at[pl.ds(n_full * _H, tail_h), pl.ds(base, b_per_w)],
                slab_v.at[b, pl.ds(0, tail_h), :], sems[b])

        for b in range(_RING):
            fire(b, b)
        pltpu.sync_copy(tgt_hbm.at[pl.ds(base, b_per_w)], tgt_v)
        a_dmas = [
            pltpu.async_copy(
                alpha_hbm.at[tgt_v.at[pl.ds(j * _CHUNK, _CHUNK)]],
                a_v.at[j], sem_a)
            for j in range(n_chunks)
        ]
        row_iota = lax.iota(jnp.int32, _L)

        def wait_slab(b, h):
            pltpu.make_async_copy(
                xt_hbm.at[pl.ds(0, h), pl.ds(base, b_per_w)],
                slab_v.at[b] if h == _H else slab_v.at[b, pl.ds(0, h), :],
                sems[b]).wait()

        def process(g, b, tail):
            for s in range(n_steps):
                t16 = tgt_v[pl.ds(s * _L, _L)]
                col16 = s * _L + row_iota
                if tail:
                    row16 = jax.lax.bitwise_and(t16, tail_h - 1)
                    hit = (t16 >> 3) == (c - tail_h) // 8
                else:
                    row16 = jax.lax.bitwise_and(t16, _H - 1)
                    hit = (t16 >> 5) == g
                v16 = plsc.load_gather(slab_v.at[b], [row16, col16])
                dst_r = jnp.full((_L,), s // 8, jnp.int32)
                plsc.store_scatter(vals_v, [dst_r, (s % 8) * _L + row_iota],
                                   v16, mask=hit)

        n_loop = (n_full // _RING) * _RING

        @pl.loop(0, n_loop, step=_RING)
        def _(g0):
            for b in range(_RING):
                g = g0 + b
                wait_slab(b, _H)
                process(g, b, tail=False)
                nxt = g + _RING

                @pl.when(nxt < n_full)
                def _():
                    fire(nxt, b)

                @pl.when(nxt == n_full)
                def _():
                    fire_tail(b)

        for g in range(n_loop, n_full):
            b = g % _RING
            wait_slab(b, _H)
            process(g, b, tail=False)
        bt = n_full % _RING
        wait_slab(bt, tail_h)
        process(n_full, bt, tail=True)

        for d in a_dmas:
            d.wait()
        pltpu.sync_copy(vals_v, vals_out.at[pl.ds(wid * n_chunks, n_chunks), :])
        pltpu.sync_copy(a_v, avals_out.at[pl.ds(wid * n_chunks, n_chunks), :])

    return k(xt, targets, alpha_flat)


def _loss_body(v_ref, a_ref, o_ref, *, inv_n):
    p = jax.nn.sigmoid(v_ref[...])
    o_ref[...] = (-jnp.sum(a_ref[...] * jnp.log(p)) * inv_n).reshape(1, 1)


def kernel(inputs, targets, alpha):
    n, c = inputs.shape
    alpha_flat = alpha.reshape(-1)
    tgt = targets.astype(jnp.int32)
    xt = inputs.T

    vals, avals = _sc_gather(xt, tgt, alpha_flat, n, c)

    out = pl.pallas_call(
        functools.partial(_loss_body, inv_n=1.0 / n),
        out_shape=jax.ShapeDtypeStruct((1, 1), jnp.float32),
    )(vals, avals)
    return out[0, 0]


# speedup vs baseline: 2.8565x; 1.0014x over previous
import functools

import jax
import jax.numpy as jnp
from jax import lax
from jax.experimental import pallas as pl
from jax.experimental.pallas import tpu as pltpu
from jax.experimental.pallas import tpu_sc as plsc

_NC = 2
_NS = 16
_NW = _NC * _NS
_L = 16
_CHUNK = 128
_RING = 6
_H = 32


def _sc_gather(xt, targets, alpha_flat, n, c):
    b_per_w = n // _NW
    n_chunks = b_per_w // _CHUNK
    n_steps = b_per_w // _L
    n_full = c // _H
    tail_h = c - n_full * _H

    out_sd = jax.ShapeDtypeStruct((n // _CHUNK, _CHUNK), jnp.float32)
    mesh = plsc.VectorSubcoreMesh(core_axis_name="c", subcore_axis_name="s")

    @functools.partial(
        pl.kernel,
        out_type=(out_sd, out_sd),
        mesh=mesh,
        compiler_params=pltpu.CompilerParams(
            use_tc_tiling_on_sc=True, needs_layout_passes=False),
        scratch_types=[
            pltpu.VMEM((b_per_w,), jnp.int32),
            pltpu.VMEM((n_chunks, _CHUNK), jnp.float32),
            pltpu.VMEM((n_chunks, _CHUNK), jnp.float32),
            pltpu.VMEM((_RING, _H, b_per_w), jnp.float32),
            pltpu.SemaphoreType.DMA,
            pltpu.SemaphoreType.DMA,
            pltpu.SemaphoreType.DMA,
            pltpu.SemaphoreType.DMA,
            pltpu.SemaphoreType.DMA,
            pltpu.SemaphoreType.DMA,
            pltpu.SemaphoreType.DMA,
        ],
    )
    def k(xt_hbm, tgt_hbm, alpha_hbm, vals_out, avals_out,
          tgt_v, a_v, vals_v, slab_v, s0, s1, s2, s3, s4, s5, sem_a):
        sems = (s0, s1, s2, s3, s4, s5)
        wid = lax.axis_index("s") * _NC + lax.axis_index("c")
        base = wid * b_per_w

        def fire(g, b):
            pltpu.async_copy(
                xt_hbm.at[pl.ds(g * _H, _H), pl.ds(base, b_per_w)],
                slab_v.at[b], sems[b])

        def fire_tail(b):
            pltpu.async_copy(
                xt_hbm.at[pl.ds(n_full * _H, tail_h), pl.ds(base, b_per_w)],
                slab_v.at[b, pl.ds(0, tail_h), :], sems[b])

        for b in range(_RING):
            fire(b, b)
        pltpu.sync_copy(tgt_hbm.at[pl.ds(base, b_per_w)], tgt_v)
        a_dmas = [
            pltpu.async_copy(
                alpha_hbm.at[tgt_v.at[pl.ds(j * _CHUNK, _CHUNK)]],
                a_v.at[j], sem_a)
            for j in range(n_chunks)
        ]
        row_iota = lax.iota(jnp.int32, _L)

        def wait_slab(b, h):
            pltpu.make_async_copy(
                xt_hbm.at[pl.ds(0, h), pl.ds(base, b_per_w)],
                slab_v.at[b] if h == _H else slab_v.at[b, pl.ds(0, h), :],
                sems[b]).wait()

        def process(g, b, tail):
            for s in range(n_steps):
                t16 = tgt_v[pl.ds(s * _L, _L)]
                col16 = s * _L + row_iota
                if tail:
                    row16 = jax.lax.bitwise_and(t16, tail_h - 1)
                    hit = (t16 >> 3) == (c - tail_h) // 8
                else:
                    row16 = jax.lax.bitwise_and(t16, _H - 1)
                    hit = (t16 >> 5) == g
                v16 = plsc.load_gather(slab_v.at[b], [row16, col16])
                dst_r = jnp.full((_L,), s // 8, jnp.int32)
                plsc.store_scatter(vals_v, [dst_r, (s % 8) * _L + row_iota],
                                   v16, mask=hit)

        n_loop = (n_full // _RING) * _RING

        @pl.loop(0, n_loop, step=_RING)
        def _(g0):
            for b in range(_RING):
                g = g0 + b
                wait_slab(b, _H)
                process(g, b, tail=False)
                nxt = g + _RING

                @pl.when(nxt < n_full)
                def _():
                    fire(nxt, b)

                @pl.when(nxt == n_full)
                def _():
                    fire_tail(b)

        for g in range(n_loop, n_full):
            b = g % _RING
            wait_slab(b, _H)
            process(g, b, tail=False)
        bt = n_full % _RING
        wait_slab(bt, tail_h)
        process(n_full, bt, tail=True)

        for d in a_dmas:
            d.wait()
        pltpu.sync_copy(vals_v, vals_out.at[pl.ds(wid * n_chunks, n_chunks), :])
        pltpu.sync_copy(a_v, avals_out.at[pl.ds(wid * n_chunks, n_chunks), :])

    return k(xt, targets, alpha_flat)


def _loss_body(v_ref, a_ref, o_ref, *, inv_n):
    p = jax.nn.sigmoid(v_ref[...])
    o_ref[...] = (-jnp.sum(a_ref[...] * jnp.log(p)) * inv_n).reshape(1, 1)


def kernel(inputs, targets, alpha):
    n, c = inputs.shape
    alpha_flat = alpha.reshape(-1)
    tgt = targets.astype(jnp.int32)
    xt = inputs.T

    vals, avals = _sc_gather(xt, tgt, alpha_flat, n, c)

    out = pl.pallas_call(
        functools.partial(_loss_body, inv_n=1.0 / n),
        out_shape=jax.ShapeDtypeStruct((1, 1), jnp.float32),
    )(vals, avals)
    return out[0, 0]
